# baseline (speedup 1.0000x reference)
import functools

import jax
import jax.numpy as jnp
from jax import lax
from jax.experimental import pallas as pl
from jax.experimental.pallas import tpu as pltpu
from jax.experimental.pallas import tpu_sc as plsc


def _scale_body(t_ref, w_ref, o_ref):
    o_ref[...] = t_ref[...] * w_ref[pl.program_id(0), 0]


def _scale_tables(tables2d, w_feat, nd, v, d):
    return pl.pallas_call(
        _scale_body,
        grid=(nd,),
        in_specs=[
            pl.BlockSpec((v, d), lambda f: (f, 0)),
            pl.BlockSpec(memory_space=pltpu.SMEM),
        ],
        out_specs=pl.BlockSpec((v, d), lambda f: (f, 0)),
        out_shape=jax.ShapeDtypeStruct((nd * v, d), jnp.float32),
    )(tables2d, w_feat)


_SUB = 128


def _make_gather(nd, v, d, b, nw):
    bpw = b // nw
    nsub = bpw // _SUB
    nrow = nd * nsub
    mesh = plsc.VectorSubcoreMesh(
        core_axis_name="c", subcore_axis_name="s",
        num_cores=2, num_subcores=16,
    )
    ncores = mesh.num_cores

    @functools.partial(
        pl.kernel,
        mesh=mesh,
        out_type=jax.ShapeDtypeStruct((b, d), jnp.float32),
        scratch_types=[
            pltpu.VMEM((nrow, _SUB), jnp.int32),
            pltpu.VMEM((bpw, d), jnp.float32),
            pltpu.SemaphoreType.DMA,
        ],
    )
    def _gather(idx_hbm, st_hbm, out_hbm, idx_v, acc_v, sem):
        wid = lax.axis_index("s") * ncores + lax.axis_index("c")
        base = wid * bpw
        pltpu.sync_copy(idx_hbm.at[wid], idx_v)

        def _off_body(r, carry):
            off = (r // nsub) * v
            for j in range(_SUB // 16):
                sl = pl.ds(j * 16, 16)
                idx_v[r, sl] = idx_v[r, sl] + off
            return carry

        lax.fori_loop(0, nrow, _off_body, 0)

        def _fire0(r, carry):
            pltpu.async_copy(
                st_hbm.at[idx_v.at[r]], acc_v.at[pl.ds(r * _SUB, _SUB), :],
                sem,
            )
            return carry

        lax.fori_loop(0, nsub, _fire0, 0)
        pltpu.make_async_copy(st_hbm.at[pl.ds(0, bpw), :], acc_v, sem).wait()

        def _fire(r, carry):
            pltpu.async_copy(
                st_hbm.at[idx_v.at[r]],
                acc_v.at[pl.ds((r % nsub) * _SUB, _SUB), :],
                sem, add=True,
            )
            return carry

        lax.fori_loop(nsub, nrow, _fire, 0)

        def _drain(r, carry):
            pltpu.make_async_copy(
                st_hbm.at[pl.ds(0, bpw), :], acc_v, sem
            ).wait()
            return carry

        lax.fori_loop(0, nd - 1, _drain, 0)

        pltpu.sync_copy(acc_v, out_hbm.at[pl.ds(base, bpw), :])

    return _gather


def _finish_body(cont_ref, pre_ref, w1_ref, b1_ref, w2_ref, b2_ref,
                 w0_ref, cb_ref, g_ref, bt_ref, o_ref):
    cf = cont_ref[...]
    cf = jnp.where(jnp.isnan(cf), 0.0, cf)
    h = jnp.dot(cf, w1_ref[...], preferred_element_type=jnp.float32,
                precision=lax.Precision.HIGHEST)
    h = jnp.maximum(h + b1_ref[...], 0.0)
    h = jnp.clip(h, -65000.0, 65000.0)
    e = jnp.dot(h, w2_ref[...], preferred_element_type=jnp.float32,
                precision=lax.Precision.HIGHEST)
    e = jnp.maximum(e + b2_ref[...], 0.0)
    x = pre_ref[...] + e * w0_ref[...] + cb_ref[...]
    mu = jnp.mean(x, axis=-1, keepdims=True)
    xc = x - mu
    var = jnp.mean(xc * xc, axis=-1, keepdims=True)
    o_ref[...] = xc * lax.rsqrt(var + 1e-5) * g_ref[...] + bt_ref[...]


def _finish(cont, pre, w1, b1, w2, b2, w0, cb, gamma, beta, blk):
    b, nc = cont.shape
    d = pre.shape[1]
    nh = w1.shape[1]

    def full(shape):
        return pl.BlockSpec(shape, lambda i: (0, 0))

    return pl.pallas_call(
        _finish_body,
        grid=(b // blk,),
        in_specs=[
            pl.BlockSpec((blk, nc), lambda i: (i, 0)),
            pl.BlockSpec((blk, d), lambda i: (i, 0)),
            full((nc, nh)), full((1, nh)), full((nh, d)), full((1, d)),
            full((1, 1)), full((1, 1)), full((1, d)), full((1, d)),
        ],
        out_specs=pl.BlockSpec((blk, d), lambda i: (i, 0)),
        out_shape=jax.ShapeDtypeStruct((b, d), jnp.float32),
    )(cont, pre, w1, b1.reshape(1, nh), w2, b2.reshape(1, d),
      w0, cb, gamma.reshape(1, d), beta.reshape(1, d))


def kernel(cont, disc, W1, b1, W2, b2, tables, combine_w, combine_b,
           gamma, beta):
    b, nc = cont.shape
    nd, v, d = tables.shape
    nw = 32

    tables2d = tables.reshape(nd * v, d)
    bpw = b // nw
    nsub = bpw // _SUB
    idx_w = (
        disc.T.reshape(nd, nw, nsub, _SUB)
        .transpose(1, 0, 2, 3)
        .reshape(nw, nd * nsub, _SUB)
    )
    scaled = _scale_tables(tables2d, combine_w[1:], nd, v, d)
    pre = _make_gather(nd, v, d, b, nw)(idx_w, scaled)
    out = _finish(cont, pre, W1, b1, W2, b2,
                  combine_w[0:1], combine_b.reshape(1, 1), gamma, beta,
                  blk=1024)
    return out


# speedup vs baseline: 1.0956x; 1.0956x over previous
import functools

import jax
import jax.numpy as jnp
from jax import lax
from jax.experimental import pallas as pl
from jax.experimental.pallas import tpu as pltpu
from jax.experimental.pallas import tpu_sc as plsc


def _scale_body(t_ref, w_ref, o_ref):
    o_ref[...] = t_ref[...] * w_ref[pl.program_id(0), 0]


def _scale_tables(tables2d, w_feat, nd, v, d):
    return pl.pallas_call(
        _scale_body,
        grid=(nd,),
        in_specs=[
            pl.BlockSpec((v, d), lambda f: (f, 0)),
            pl.BlockSpec(memory_space=pltpu.SMEM),
        ],
        out_specs=pl.BlockSpec((v, d), lambda f: (f, 0)),
        out_shape=jax.ShapeDtypeStruct((nd * v, d), jnp.float32),
    )(tables2d, w_feat)


_SUB = 128


def _make_gather(nd, v, d, b, nw):
    bpw = b // nw
    nsub = bpw // _SUB
    nrow = nd * nsub
    mesh = plsc.VectorSubcoreMesh(
        core_axis_name="c", subcore_axis_name="s",
        num_cores=2, num_subcores=16,
    )
    ncores = mesh.num_cores

    @functools.partial(
        pl.kernel,
        mesh=mesh,
        out_type=jax.ShapeDtypeStruct((b, d), jnp.float32),
        scratch_types=[
            pltpu.VMEM((nrow, _SUB), jnp.int32),
            pltpu.VMEM((bpw, d), jnp.float32),
            pltpu.SemaphoreType.DMA,
        ],
    )
    def _gather(idx_hbm, st_hbm, out_hbm, idx_v, acc_v, sem):
        wid = lax.axis_index("s") * ncores + lax.axis_index("c")
        base = wid * bpw
        pltpu.sync_copy(idx_hbm.at[wid], idx_v)

        def _off_body(r, carry):
            off = (r // nsub) * v
            for j in range(_SUB // 16):
                sl = pl.ds(j * 16, 16)
                idx_v[r, sl] = idx_v[r, sl] + off
            return carry

        lax.fori_loop(0, nrow, _off_body, 0)

        def _fire0(r, carry):
            pltpu.async_copy(
                st_hbm.at[idx_v.at[r]], acc_v.at[pl.ds(r * _SUB, _SUB), :],
                sem,
            )
            return carry

        lax.fori_loop(0, nsub, _fire0, 0)
        pltpu.make_async_copy(st_hbm.at[pl.ds(0, bpw), :], acc_v, sem).wait()

        def _fire(r, carry):
            pltpu.async_copy(
                st_hbm.at[idx_v.at[r]],
                acc_v.at[pl.ds((r % nsub) * _SUB, _SUB), :],
                sem, add=True,
            )
            return carry

        lax.fori_loop(nsub, nrow, _fire, 0)

        def _drain(r, carry):
            pltpu.make_async_copy(
                st_hbm.at[pl.ds(0, bpw), :], acc_v, sem
            ).wait()
            return carry

        lax.fori_loop(0, nd - 1, _drain, 0)

        pltpu.sync_copy(acc_v, out_hbm.at[pl.ds(base, bpw), :])

    return _gather


def _finish_body(cont_ref, pre_ref, w1_ref, b1_ref, w2_ref, b2_ref,
                 w0_ref, cb_ref, g_ref, bt_ref, o_ref):
    cf = cont_ref[...]
    cf = jnp.where(jnp.isnan(cf), 0.0, cf)
    h = jnp.dot(cf, w1_ref[...], preferred_element_type=jnp.float32)
    h = jnp.maximum(h + b1_ref[...], 0.0)
    h = jnp.clip(h, -65000.0, 65000.0)
    e = jnp.dot(h, w2_ref[...], preferred_element_type=jnp.float32)
    e = jnp.maximum(e + b2_ref[...], 0.0)
    x = pre_ref[...] + e * w0_ref[...] + cb_ref[...]
    mu = jnp.mean(x, axis=-1, keepdims=True)
    xc = x - mu
    var = jnp.mean(xc * xc, axis=-1, keepdims=True)
    o_ref[...] = xc * lax.rsqrt(var + 1e-5) * g_ref[...] + bt_ref[...]


def _finish(cont, pre, w1, b1, w2, b2, w0, cb, gamma, beta, blk):
    b, nc = cont.shape
    d = pre.shape[1]
    nh = w1.shape[1]

    def full(shape):
        return pl.BlockSpec(shape, lambda i: (0, 0))

    return pl.pallas_call(
        _finish_body,
        grid=(b // blk,),
        in_specs=[
            pl.BlockSpec((blk, nc), lambda i: (i, 0)),
            pl.BlockSpec((blk, d), lambda i: (i, 0)),
            full((nc, nh)), full((1, nh)), full((nh, d)), full((1, d)),
            full((1, 1)), full((1, 1)), full((1, d)), full((1, d)),
        ],
        out_specs=pl.BlockSpec((blk, d), lambda i: (i, 0)),
        out_shape=jax.ShapeDtypeStruct((b, d), jnp.float32),
    )(cont, pre, w1, b1.reshape(1, nh), w2, b2.reshape(1, d),
      w0, cb, gamma.reshape(1, d), beta.reshape(1, d))


def kernel(cont, disc, W1, b1, W2, b2, tables, combine_w, combine_b,
           gamma, beta):
    b, nc = cont.shape
    nd, v, d = tables.shape
    nw = 32

    tables2d = tables.reshape(nd * v, d)
    bpw = b // nw
    nsub = bpw // _SUB
    idx_w = (
        disc.T.reshape(nd, nw, nsub, _SUB)
        .transpose(1, 0, 2, 3)
        .reshape(nw, nd * nsub, _SUB)
    )
    scaled = _scale_tables(tables2d, combine_w[1:], nd, v, d)
    pre = _make_gather(nd, v, d, b, nw)(idx_w, scaled)
    out = _finish(cont, pre, W1, b1, W2, b2,
                  combine_w[0:1], combine_b.reshape(1, 1), gamma, beta,
                  blk=1024)
    return out


# speedup vs baseline: 1.1840x; 1.0806x over previous
import functools

import jax
import jax.numpy as jnp
from jax import lax
from jax.experimental import pallas as pl
from jax.experimental.pallas import tpu as pltpu
from jax.experimental.pallas import tpu_sc as plsc


def _scale_body(t_ref, w_ref, o_ref):
    o_ref[...] = t_ref[...] * w_ref[pl.program_id(0), 0]


def _scale_tables(tables2d, w_feat, nd, v, d):
    return pl.pallas_call(
        _scale_body,
        grid=(nd,),
        in_specs=[
            pl.BlockSpec((v, d), lambda f: (f, 0)),
            pl.BlockSpec(memory_space=pltpu.SMEM),
        ],
        out_specs=pl.BlockSpec((v, d), lambda f: (f, 0)),
        out_shape=jax.ShapeDtypeStruct((nd * v, d), jnp.float32),
    )(tables2d, w_feat)


_SUB = 128


def _make_gather(nd, v, d, b, nw):
    bpw = b // nw
    nsub = bpw // _SUB
    nrow = nd * nsub
    mesh = plsc.VectorSubcoreMesh(
        core_axis_name="c", subcore_axis_name="s",
        num_cores=2, num_subcores=16,
    )
    ncores = mesh.num_cores

    assert nsub == 4
    scratch = [
        pltpu.VMEM((nrow, _SUB), jnp.int32),
        pltpu.VMEM((bpw, d), jnp.float32),
        pltpu.SemaphoreType.DMA,
        pltpu.SemaphoreType.DMA,
        pltpu.SemaphoreType.DMA,
        pltpu.SemaphoreType.DMA,
    ]

    @functools.partial(
        pl.kernel,
        mesh=mesh,
        out_type=jax.ShapeDtypeStruct((b, d), jnp.float32),
        scratch_types=scratch,
    )
    def _gather(idx_hbm, st_hbm, out_hbm, idx_v, acc_v, s0, s1, s2, s3):
        sems = (s0, s1, s2, s3)
        wid = lax.axis_index("s") * ncores + lax.axis_index("c")
        base = wid * bpw
        pltpu.sync_copy(idx_hbm.at[wid], idx_v)

        def _off_body(r, carry):
            off = (r // nsub) * v
            for j in range(_SUB // 16):
                sl = pl.ds(j * 16, 16)
                idx_v[r, sl] = idx_v[r, sl] + off
            return carry

        lax.fori_loop(0, nrow, _off_body, 0)

        def _acc_at(c):
            return acc_v.at[pl.ds(c * _SUB, _SUB), :]

        dummy_src = st_hbm.at[pl.ds(0, _SUB), :]

        for c in range(nsub):
            pltpu.async_copy(st_hbm.at[idx_v.at[c]], _acc_at(c), sems[c])

        for c in range(nsub):
            pltpu.make_async_copy(dummy_src, _acc_at(c), sems[c]).wait()

            def _fire(f, carry, c=c):
                pltpu.async_copy(
                    st_hbm.at[idx_v.at[f * nsub + c]], _acc_at(c),
                    sems[c], add=True,
                )
                return carry

            lax.fori_loop(1, nd, _fire, 0)

        for c in range(nsub):
            def _drain(f, carry, c=c):
                pltpu.make_async_copy(dummy_src, _acc_at(c), sems[c]).wait()
                return carry

            lax.fori_loop(1, nd, _drain, 0)
            pltpu.sync_copy(
                acc_v.at[pl.ds(c * _SUB, _SUB), :],
                out_hbm.at[pl.ds(base + c * _SUB, _SUB), :],
            )

    return _gather


def _finish_body(cont_ref, pre_ref, w1_ref, b1_ref, w2_ref, b2_ref,
                 w0_ref, cb_ref, g_ref, bt_ref, o_ref):
    cf = cont_ref[...]
    cf = jnp.where(jnp.isnan(cf), 0.0, cf)
    h = jnp.dot(cf, w1_ref[...], preferred_element_type=jnp.float32)
    h = jnp.maximum(h + b1_ref[...], 0.0)
    h = jnp.clip(h, -65000.0, 65000.0)
    e = jnp.dot(h, w2_ref[...], preferred_element_type=jnp.float32)
    e = jnp.maximum(e + b2_ref[...], 0.0)
    x = pre_ref[...] + e * w0_ref[...] + cb_ref[...]
    mu = jnp.mean(x, axis=-1, keepdims=True)
    xc = x - mu
    var = jnp.mean(xc * xc, axis=-1, keepdims=True)
    o_ref[...] = xc * lax.rsqrt(var + 1e-5) * g_ref[...] + bt_ref[...]


def _finish(cont, pre, w1, b1, w2, b2, w0, cb, gamma, beta, blk):
    b, nc = cont.shape
    d = pre.shape[1]
    nh = w1.shape[1]

    def full(shape):
        return pl.BlockSpec(shape, lambda i: (0, 0))

    return pl.pallas_call(
        _finish_body,
        grid=(b // blk,),
        in_specs=[
            pl.BlockSpec((blk, nc), lambda i: (i, 0)),
            pl.BlockSpec((blk, d), lambda i: (i, 0)),
            full((nc, nh)), full((1, nh)), full((nh, d)), full((1, d)),
            full((1, 1)), full((1, 1)), full((1, d)), full((1, d)),
        ],
        out_specs=pl.BlockSpec((blk, d), lambda i: (i, 0)),
        out_shape=jax.ShapeDtypeStruct((b, d), jnp.float32),
    )(cont, pre, w1, b1.reshape(1, nh), w2, b2.reshape(1, d),
      w0, cb, gamma.reshape(1, d), beta.reshape(1, d))


def kernel(cont, disc, W1, b1, W2, b2, tables, combine_w, combine_b,
           gamma, beta):
    b, nc = cont.shape
    nd, v, d = tables.shape
    nw = 32

    tables2d = tables.reshape(nd * v, d)
    bpw = b // nw
    nsub = bpw // _SUB
    idx_w = (
        disc.T.reshape(nd, nw, nsub, _SUB)
        .transpose(1, 0, 2, 3)
        .reshape(nw, nd * nsub, _SUB)
    )
    scaled = _scale_tables(tables2d, combine_w[1:], nd, v, d)
    pre = _make_gather(nd, v, d, b, nw)(idx_w, scaled)
    out = _finish(cont, pre, W1, b1, W2, b2,
                  combine_w[0:1], combine_b.reshape(1, 1), gamma, beta,
                  blk=1024)
    return out
